# baseline (speedup 1.0000x reference)
import functools

import jax
import jax.numpy as jnp
from jax import lax
from jax.experimental import pallas as pl
from jax.experimental.pallas import tpu as pltpu
from jax.experimental.pallas import tpu_sc as plsc

F32 = jnp.float32

K = 128


def _dot(a, b):
    return jax.lax.dot_general(a, b, (((1,), (0,)), ((), ())),
                               preferred_element_type=F32)


def _edge_prep_body(ef_ref, w1, b1, w2, b2, wl, bl, out_ref):
    x2 = ef_ref[...]
    din = x2.shape[1] // 2
    outs = []
    for h in range(2):
        x = x2[:, h * din:(h + 1) * din]
        hh = jnp.maximum(_dot(x, w1[...]) + b1[...], 0.0)
        outs.append(_dot(hh, w2[...]) + b2[...] + _dot(x, wl[...]) + bl[...])
    out_ref[...] = jnp.concatenate(outs, axis=1)


def _edge_prep(ef2, p_ff, p_lin, block):
    e2, din2 = ef2.shape
    dout = p_lin["W"].shape[1]
    grid = e2 // block
    full = lambda arr: pl.BlockSpec(arr.shape, lambda i: (0,) * arr.ndim)
    args = (p_ff["W1"], p_ff["b1"].reshape(1, -1), p_ff["W2"],
            p_ff["b2"].reshape(1, -1), p_lin["W"], p_lin["b"].reshape(1, -1))
    return pl.pallas_call(
        _edge_prep_body,
        grid=(grid,),
        in_specs=[pl.BlockSpec((block, din2), lambda i: (i, 0))]
        + [full(a) for a in args],
        out_specs=pl.BlockSpec((block, 2 * dout), lambda i: (i, 0)),
        out_shape=jax.ShapeDtypeStruct((e2, 2 * dout), F32),
    )(ef2, *args)


def _node_prep_body(nf_ref, w1, b1, w2, b2, wl, bl, wmsg, bmsg,
                    res_ref, mn_ref, xm_ref):
    x = nf_ref[...]
    h = jnp.maximum(_dot(x, w1[...]) + b1[...], 0.0)
    res = _dot(h, w2[...]) + b2[...]
    mn = res + _dot(x, wl[...]) + bl[...]
    res_ref[...] = res
    mn_ref[...] = mn
    xm = _dot(mn, wmsg[...]) + bmsg[...]
    xm_ref[...] = jnp.concatenate([xm, jnp.zeros_like(xm)], axis=1)


def _node_prep(node_feature, p_ff, p_lin, wmsg, bmsg, block):
    n, din = node_feature.shape
    dout = p_lin["W"].shape[1]
    grid = n // block
    full = lambda arr: pl.BlockSpec(arr.shape, lambda i: (0,) * arr.ndim)
    args = (p_ff["W1"], p_ff["b1"].reshape(1, -1), p_ff["W2"],
            p_ff["b2"].reshape(1, -1), p_lin["W"], p_lin["b"].reshape(1, -1),
            wmsg, bmsg.reshape(1, -1))
    shp = jax.ShapeDtypeStruct((n, dout), F32)
    return pl.pallas_call(
        _node_prep_body,
        grid=(grid,),
        in_specs=[pl.BlockSpec((block, din), lambda i: (i, 0))]
        + [full(a) for a in args],
        out_specs=[pl.BlockSpec((block, dout), lambda i: (i, 0))] * 2
        + [pl.BlockSpec((block, 2 * dout), lambda i: (i, 0))],
        out_shape=[shp, shp, jax.ShapeDtypeStruct((n, 2 * dout), F32)],
    )(node_feature, *args)


def _state_prep_body(gs_ref, w1, b1, w2, b2, wl, bl, out_ref):
    x = gs_ref[...]
    h = jnp.maximum(_dot(x, w1[...]) + b1[...], 0.0)
    out_ref[...] = _dot(h, w2[...]) + b2[...] + _dot(x, wl[...]) + bl[...]


def _state_prep(gs_row, p_ff, p_lin):
    b = gs_row.shape[1]
    args = (p_ff["W1"], p_ff["b1"].reshape(1, -1), p_ff["W2"],
            p_ff["b2"].reshape(1, -1), p_lin["W"], p_lin["b"].reshape(1, -1))
    return pl.pallas_call(
        _state_prep_body,
        out_shape=jax.ShapeDtypeStruct((1, b), F32),
    )(gs_row, *args)


def _update_body(x_ref, a0_ref, a1_ref, res_ref, st_ref,
                 wself, wagg, wstate, bout, wmsg, bmsg,
                 xn_ref, xmn_ref):
    x = x_ref[...]
    d = x.shape[1]
    agg = a0_ref[:, :d] + a1_ref[:, :d]
    t = (_dot(x, wself[...]) + _dot(agg, wagg[...])
         + st_ref[...] * wstate[...] + bout[...])
    xn = res_ref[...] + jnp.maximum(t, 0.0)
    xn_ref[...] = xn
    xm = _dot(xn, wmsg[...]) + bmsg[...]
    xmn_ref[...] = jnp.concatenate([xm, jnp.zeros_like(xm)], axis=1)


def _update(x, a0, a1, node_res, state_col, p, wmsg_next, bmsg_next, block):
    n, d = x.shape
    grid = n // block
    full = lambda arr: pl.BlockSpec(arr.shape, lambda i: (0,) * arr.ndim)
    args = (p["Wself"], p["Wagg"], p["wstate"].reshape(1, -1),
            p["bout"].reshape(1, -1), wmsg_next, bmsg_next.reshape(1, -1))
    blk = lambda w=d: pl.BlockSpec((block, w), lambda i: (i, 0))
    return pl.pallas_call(
        _update_body,
        grid=(grid,),
        in_specs=[blk(), blk(2 * d), blk(2 * d), blk(),
                  pl.BlockSpec((block, 1), lambda i: (i, 0))]
        + [full(a) for a in args],
        out_specs=[blk(), blk(2 * d)],
        out_shape=[jax.ShapeDtypeStruct((n, d), F32),
                   jax.ShapeDtypeStruct((n, 2 * d), F32)],
    )(x, a0, a1, node_res, state_col, *args)


def _sc_msgpass(xm, me2, src, dst, wedge):
    n, dw = xm.shape
    d = dw // 2
    e = src.shape[0]
    kc = 80
    info = plsc.get_sparse_core_info()
    nc, ns = info.num_cores, info.num_subcores
    ts = e // (nc * ns * kc)
    kh = kc // 2
    nb = 2
    cp_sub = 5
    rows_w = n // cp_sub
    zr = kc
    nz = rows_w // zr

    mesh = plsc.VectorSubcoreMesh(core_axis_name="c", subcore_axis_name="s")

    @functools.partial(
        pl.kernel,
        out_type=jax.ShapeDtypeStruct((nc, n, dw), F32),
        mesh=mesh,
        scratch_types=[
            pltpu.VMEM_SHARED((n, dw), F32),
            [pltpu.VMEM((kc,), jnp.int32)] * nb,
            [pltpu.VMEM((kc,), jnp.int32)] * nb,
            [pltpu.VMEM((kc, dw), F32)] * nb,
            [pltpu.VMEM((kh, dw), F32)] * nb,
            pltpu.VMEM((d,), F32),
            [pltpu.SemaphoreType.DMA] * nb,
            [pltpu.SemaphoreType.DMA] * nb,
        ],
    )
    def k(xm_h, me_h, src_h, dst_h, wedge_h, out_h,
          agg_sh, src2, dst2, gbuf, ebuf, wv, sem_g, sem_e):
        c = lax.axis_index("c")
        s = lax.axis_index("s")

        pltpu.sync_copy(wedge_h, wv)
        w = [wv[pl.ds(j * 16, 16)] for j in range(d // 16)]
        zero = jnp.zeros((16,), F32)

        e0 = pl.multiple_of((c * ns + s) * (ts * kc), kc)
        me0 = pl.multiple_of((c * ns + s) * (ts * kh), kh)

        @pl.loop(0, kc)
        def _(r):
            for j in range(dw // 16):
                gbuf[0][r, pl.ds(j * 16, 16)] = zero

        @pl.when(s < cp_sub)
        def _():
            @pl.loop(0, nz)
            def _(kz):
                r0 = pl.multiple_of(s * rows_w + kz * zr, zr)
                pltpu.sync_copy(gbuf[0], agg_sh.at[pl.ds(r0, zr)])

        plsc.subcore_barrier()

        def body(t, nbi):
            for i in range(nbi):
                u = t + i
                eo = pl.multiple_of(e0 + u * kc, kc)
                pltpu.sync_copy(src_h.at[pl.ds(eo, kc)], src2[i])
                pltpu.sync_copy(dst_h.at[pl.ds(eo, kc)], dst2[i])
            loads = []
            for i in range(nbi):
                u = t + i
                cpg = pltpu.async_copy(xm_h.at[src2[i]], gbuf[i],
                                       sem_g[i])
                off2 = pl.multiple_of(me0 + u * kh, kh)
                cpe = pltpu.async_copy(me_h.at[pl.ds(off2, kh)], ebuf[i],
                                       sem_e[i])
                loads.append((cpg, cpe))
            for i in range(nbi):
                loads[i][0].wait()
                loads[i][1].wait()

                @pl.loop(0, kh)
                def _(rr):
                    for half in range(2):
                        r = 2 * rr + half
                        for j in range(d // 16):
                            gsl = pl.ds(j * 16, 16)
                            esl = pl.ds(half * d + j * 16, 16)
                            gbuf[i][r, gsl] = jnp.maximum(
                                gbuf[i][r, gsl] + ebuf[i][rr, esl] * w[j],
                                0.0)

                pltpu.sync_copy(gbuf[i], agg_sh.at[dst2[i]], add=True)

        main = (ts // nb) * nb
        @pl.loop(0, main, step=nb)
        def _(t):
            body(t, nb)

        for u in range(main, ts):
            body(u, 1)

        plsc.subcore_barrier()

        @pl.when(s < cp_sub)
        def _():
            @pl.loop(0, nz)
            def _(kz):
                r0 = pl.multiple_of(s * rows_w + kz * zr, zr)
                pltpu.sync_copy(agg_sh.at[pl.ds(r0, zr)], gbuf[0])
                pltpu.sync_copy(gbuf[0], out_h.at[c].at[pl.ds(r0, zr)])

    return k(xm, me2, src, dst, wedge)


def kernel(node_feature, edge_index, edge_feature, global_state, group_size,
           params):
    n = node_feature.shape[0]
    batch = global_state.shape[0]
    group = n // batch
    convs = params["convs"]

    ef2 = edge_feature.reshape(edge_feature.shape[0] // 2,
                               2 * edge_feature.shape[1])
    me2 = _edge_prep(ef2, params["edge_ff"], params["edge_linear"],
                     block=8000)
    node_res, meta_node, xm = _node_prep(
        node_feature, params["node_ff"], params["node_linear"],
        convs[0]["Wmsg"], convs[0]["bmsg"], block=2000)
    tot_state = _state_prep(global_state.reshape(1, batch),
                            params["state_ff"], params["state_linear"])
    state_col = jnp.broadcast_to(tot_state.reshape(batch, 1, 1),
                                 (batch, group, 1)).reshape(n, 1)

    src = edge_index[0]
    dst = edge_index[1]
    nl = len(convs)
    for l, p in enumerate(convs):
        agg2 = _sc_msgpass(xm, me2, src, dst, p["wedge"])
        pn = convs[(l + 1) % nl]
        meta_node, xm = _update(meta_node, agg2[0], agg2[1], node_res,
                                state_col, p, pn["Wmsg"], pn["bmsg"],
                                block=2000)
    return meta_node


# speedup vs baseline: 1.0410x; 1.0410x over previous
import functools

import jax
import jax.numpy as jnp
from jax import lax
from jax.experimental import pallas as pl
from jax.experimental.pallas import tpu as pltpu
from jax.experimental.pallas import tpu_sc as plsc

F32 = jnp.float32

K = 128


def _dot(a, b):
    return jax.lax.dot_general(a, b, (((1,), (0,)), ((), ())),
                               preferred_element_type=F32)


def _edge_prep_body(ef_ref, w1, b1, w2, b2, wl, bl, out_ref):
    x2 = ef_ref[...]
    din = x2.shape[1] // 2
    outs = []
    for h in range(2):
        x = x2[:, h * din:(h + 1) * din]
        hh = jnp.maximum(_dot(x, w1[...]) + b1[...], 0.0)
        outs.append(_dot(hh, w2[...]) + b2[...] + _dot(x, wl[...]) + bl[...])
    out_ref[...] = jnp.concatenate(outs, axis=1)


def _edge_prep(ef2, p_ff, p_lin, block):
    e2, din2 = ef2.shape
    dout = p_lin["W"].shape[1]
    grid = e2 // block
    full = lambda arr: pl.BlockSpec(arr.shape, lambda i: (0,) * arr.ndim)
    args = (p_ff["W1"], p_ff["b1"].reshape(1, -1), p_ff["W2"],
            p_ff["b2"].reshape(1, -1), p_lin["W"], p_lin["b"].reshape(1, -1))
    return pl.pallas_call(
        _edge_prep_body,
        grid=(grid,),
        in_specs=[pl.BlockSpec((block, din2), lambda i: (i, 0))]
        + [full(a) for a in args],
        out_specs=pl.BlockSpec((block, 2 * dout), lambda i: (i, 0)),
        out_shape=jax.ShapeDtypeStruct((e2, 2 * dout), F32),
    )(ef2, *args)


def _node_prep_body(nf_ref, w1, b1, w2, b2, wl, bl, wmsg, bmsg,
                    res_ref, mn_ref, xm_ref):
    x = nf_ref[...]
    h = jnp.maximum(_dot(x, w1[...]) + b1[...], 0.0)
    res = _dot(h, w2[...]) + b2[...]
    mn = res + _dot(x, wl[...]) + bl[...]
    res_ref[...] = res
    mn_ref[...] = mn
    xm = _dot(mn, wmsg[...]) + bmsg[...]
    xm_ref[...] = jnp.concatenate([xm, jnp.zeros_like(xm)], axis=1)


def _node_prep(node_feature, p_ff, p_lin, wmsg, bmsg, block):
    n, din = node_feature.shape
    dout = p_lin["W"].shape[1]
    grid = n // block
    full = lambda arr: pl.BlockSpec(arr.shape, lambda i: (0,) * arr.ndim)
    args = (p_ff["W1"], p_ff["b1"].reshape(1, -1), p_ff["W2"],
            p_ff["b2"].reshape(1, -1), p_lin["W"], p_lin["b"].reshape(1, -1),
            wmsg, bmsg.reshape(1, -1))
    shp = jax.ShapeDtypeStruct((n, dout), F32)
    return pl.pallas_call(
        _node_prep_body,
        grid=(grid,),
        in_specs=[pl.BlockSpec((block, din), lambda i: (i, 0))]
        + [full(a) for a in args],
        out_specs=[pl.BlockSpec((block, dout), lambda i: (i, 0))] * 2
        + [pl.BlockSpec((block, 2 * dout), lambda i: (i, 0))],
        out_shape=[shp, shp, jax.ShapeDtypeStruct((n, 2 * dout), F32)],
    )(node_feature, *args)


def _state_prep_body(gs_ref, w1, b1, w2, b2, wl, bl, out_ref):
    x = gs_ref[...]
    h = jnp.maximum(_dot(x, w1[...]) + b1[...], 0.0)
    out_ref[...] = _dot(h, w2[...]) + b2[...] + _dot(x, wl[...]) + bl[...]


def _state_prep(gs_row, p_ff, p_lin):
    b = gs_row.shape[1]
    args = (p_ff["W1"], p_ff["b1"].reshape(1, -1), p_ff["W2"],
            p_ff["b2"].reshape(1, -1), p_lin["W"], p_lin["b"].reshape(1, -1))
    return pl.pallas_call(
        _state_prep_body,
        out_shape=jax.ShapeDtypeStruct((1, b), F32),
    )(gs_row, *args)


def _update_body(x_ref, a0_ref, a1_ref, res_ref, st_ref,
                 wself, wagg, wstate, bout, wmsg, bmsg,
                 xn_ref, xmn_ref):
    x = x_ref[...]
    d = x.shape[1]
    agg = a0_ref[:, :d] + a1_ref[:, :d]
    t = (_dot(x, wself[...]) + _dot(agg, wagg[...])
         + st_ref[...] * wstate[...] + bout[...])
    xn = res_ref[...] + jnp.maximum(t, 0.0)
    xn_ref[...] = xn
    xm = _dot(xn, wmsg[...]) + bmsg[...]
    xmn_ref[...] = jnp.concatenate([xm, jnp.zeros_like(xm)], axis=1)


def _update(x, a0, a1, node_res, state_col, p, wmsg_next, bmsg_next, block):
    n, d = x.shape
    grid = n // block
    full = lambda arr: pl.BlockSpec(arr.shape, lambda i: (0,) * arr.ndim)
    args = (p["Wself"], p["Wagg"], p["wstate"].reshape(1, -1),
            p["bout"].reshape(1, -1), wmsg_next, bmsg_next.reshape(1, -1))
    blk = lambda w=d: pl.BlockSpec((block, w), lambda i: (i, 0))
    return pl.pallas_call(
        _update_body,
        grid=(grid,),
        in_specs=[blk(), blk(2 * d), blk(2 * d), blk(),
                  pl.BlockSpec((block, 1), lambda i: (i, 0))]
        + [full(a) for a in args],
        out_specs=[blk(), blk(2 * d)],
        out_shape=[jax.ShapeDtypeStruct((n, d), F32),
                   jax.ShapeDtypeStruct((n, 2 * d), F32)],
    )(x, a0, a1, node_res, state_col, *args)


def _sc_msgpass(xm, me2, src, dst, wedge):
    n, dw = xm.shape
    d = dw // 2
    e = src.shape[0]
    kc = 80
    info = plsc.get_sparse_core_info()
    nc, ns = info.num_cores, info.num_subcores
    ts = e // (nc * ns * kc)
    kh = kc // 2
    nb = 2
    cp_sub = 5
    rows_w = n // cp_sub
    zr = kc
    nz = rows_w // zr

    mesh = plsc.VectorSubcoreMesh(core_axis_name="c", subcore_axis_name="s")

    @functools.partial(
        pl.kernel,
        out_type=jax.ShapeDtypeStruct((nc, n, dw), F32),
        mesh=mesh,
        scratch_types=[
            pltpu.VMEM_SHARED((n, dw), F32),
            [pltpu.VMEM((kc,), jnp.int32)] * nb,
            [pltpu.VMEM((kc,), jnp.int32)] * nb,
            [pltpu.VMEM((kc, dw), F32)] * nb,
            [pltpu.VMEM((kh, dw), F32)] * nb,
            pltpu.VMEM((d,), F32),
            [pltpu.SemaphoreType.DMA] * nb,
            [pltpu.SemaphoreType.DMA] * nb,
            [pltpu.SemaphoreType.DMA] * nb,
        ],
    )
    def k(xm_h, me_h, src_h, dst_h, wedge_h, out_h,
          agg_sh, src2, dst2, gbuf, ebuf, wv, sem_g, sem_e, sem_sc):
        c = lax.axis_index("c")
        s = lax.axis_index("s")

        pltpu.sync_copy(wedge_h, wv)
        w = [wv[pl.ds(j * 16, 16)] for j in range(d // 16)]
        zero = jnp.zeros((16,), F32)

        e0 = pl.multiple_of((c * ns + s) * (ts * kc), kc)
        me0 = pl.multiple_of((c * ns + s) * (ts * kh), kh)

        @pl.loop(0, kc)
        def _(r):
            for j in range(dw // 16):
                gbuf[0][r, pl.ds(j * 16, 16)] = zero

        @pl.when(s < cp_sub)
        def _():
            @pl.loop(0, nz)
            def _(kz):
                r0 = pl.multiple_of(s * rows_w + kz * zr, zr)
                pltpu.sync_copy(gbuf[0], agg_sh.at[pl.ds(r0, zr)])

        plsc.subcore_barrier()

        def body(t, nbi):
            for i in range(nbi):
                u = t + i
                eo = pl.multiple_of(e0 + u * kc, kc)
                pltpu.sync_copy(src_h.at[pl.ds(eo, kc)], src2[i])
                pltpu.sync_copy(dst_h.at[pl.ds(eo, kc)], dst2[i])
            loads = []
            for i in range(nbi):
                u = t + i
                cpg = pltpu.async_copy(xm_h.at[src2[i]], gbuf[i],
                                       sem_g[i])
                off2 = pl.multiple_of(me0 + u * kh, kh)
                cpe = pltpu.async_copy(me_h.at[pl.ds(off2, kh)], ebuf[i],
                                       sem_e[i])
                loads.append((cpg, cpe))
            stores = []
            for i in range(nbi):
                loads[i][0].wait()
                loads[i][1].wait()

                @pl.loop(0, kh)
                def _(rr):
                    for half in range(2):
                        r = 2 * rr + half
                        for j in range(d // 16):
                            gsl = pl.ds(j * 16, 16)
                            esl = pl.ds(half * d + j * 16, 16)
                            gbuf[i][r, gsl] = jnp.maximum(
                                gbuf[i][r, gsl] + ebuf[i][rr, esl] * w[j],
                                0.0)

                stores.append(pltpu.async_copy(
                    gbuf[i], agg_sh.at[dst2[i]], sem_sc[i], add=True))
            for cp in stores:
                cp.wait()

        main = (ts // nb) * nb
        @pl.loop(0, main, step=nb)
        def _(t):
            body(t, nb)

        for u in range(main, ts):
            body(u, 1)

        plsc.subcore_barrier()

        @pl.when(s < cp_sub)
        def _():
            @pl.loop(0, nz)
            def _(kz):
                r0 = pl.multiple_of(s * rows_w + kz * zr, zr)
                pltpu.sync_copy(agg_sh.at[pl.ds(r0, zr)], gbuf[0])
                pltpu.sync_copy(gbuf[0], out_h.at[c].at[pl.ds(r0, zr)])

    return k(xm, me2, src, dst, wedge)


def kernel(node_feature, edge_index, edge_feature, global_state, group_size,
           params):
    n = node_feature.shape[0]
    batch = global_state.shape[0]
    group = n // batch
    convs = params["convs"]

    ef2 = edge_feature.reshape(edge_feature.shape[0] // 2,
                               2 * edge_feature.shape[1])
    me2 = _edge_prep(ef2, params["edge_ff"], params["edge_linear"],
                     block=8000)
    node_res, meta_node, xm = _node_prep(
        node_feature, params["node_ff"], params["node_linear"],
        convs[0]["Wmsg"], convs[0]["bmsg"], block=2000)
    tot_state = _state_prep(global_state.reshape(1, batch),
                            params["state_ff"], params["state_linear"])
    state_col = jnp.broadcast_to(tot_state.reshape(batch, 1, 1),
                                 (batch, group, 1)).reshape(n, 1)

    src = edge_index[0]
    dst = edge_index[1]
    nl = len(convs)
    for l, p in enumerate(convs):
        agg2 = _sc_msgpass(xm, me2, src, dst, p["wedge"])
        pn = convs[(l + 1) % nl]
        meta_node, xm = _update(meta_node, agg2[0], agg2[1], node_res,
                                state_col, p, pn["Wmsg"], pn["bmsg"],
                                block=2000)
    return meta_node


# speedup vs baseline: 1.2448x; 1.1958x over previous
import functools

import jax
import jax.numpy as jnp
from jax import lax
from jax.experimental import pallas as pl
from jax.experimental.pallas import tpu as pltpu
from jax.experimental.pallas import tpu_sc as plsc

F32 = jnp.float32

K = 128


def _dot(a, b):
    return jax.lax.dot_general(a, b, (((1,), (0,)), ((), ())),
                               preferred_element_type=F32)


def _edge_prep_body(ef_ref, w1, b1, w2, b2, wl, bl, out_ref):
    x2 = ef_ref[...]
    din = x2.shape[1] // 2
    outs = []
    for h in range(2):
        x = x2[:, h * din:(h + 1) * din]
        hh = jnp.maximum(_dot(x, w1[...]) + b1[...], 0.0)
        outs.append(_dot(hh, w2[...]) + b2[...] + _dot(x, wl[...]) + bl[...])
    out_ref[...] = jnp.concatenate(outs, axis=1)


def _edge_prep(ef2, p_ff, p_lin, block):
    e2, din2 = ef2.shape
    dout = p_lin["W"].shape[1]
    grid = e2 // block
    full = lambda arr: pl.BlockSpec(arr.shape, lambda i: (0,) * arr.ndim)
    args = (p_ff["W1"], p_ff["b1"].reshape(1, -1), p_ff["W2"],
            p_ff["b2"].reshape(1, -1), p_lin["W"], p_lin["b"].reshape(1, -1))
    return pl.pallas_call(
        _edge_prep_body,
        grid=(grid,),
        in_specs=[pl.BlockSpec((block, din2), lambda i: (i, 0))]
        + [full(a) for a in args],
        out_specs=pl.BlockSpec((block, 2 * dout), lambda i: (i, 0)),
        out_shape=jax.ShapeDtypeStruct((e2, 2 * dout), F32),
    )(ef2, *args)


def _node_prep_body(nf_ref, w1, b1, w2, b2, wl, bl, wmsg, bmsg,
                    res_ref, mn_ref, xm_ref):
    x = nf_ref[...]
    h = jnp.maximum(_dot(x, w1[...]) + b1[...], 0.0)
    res = _dot(h, w2[...]) + b2[...]
    mn = res + _dot(x, wl[...]) + bl[...]
    res_ref[...] = res
    mn_ref[...] = mn
    xm = _dot(mn, wmsg[...]) + bmsg[...]
    xm_ref[...] = jnp.concatenate([xm, jnp.zeros_like(xm)], axis=1)


def _node_prep(node_feature, p_ff, p_lin, wmsg, bmsg, block):
    n, din = node_feature.shape
    dout = p_lin["W"].shape[1]
    grid = n // block
    full = lambda arr: pl.BlockSpec(arr.shape, lambda i: (0,) * arr.ndim)
    args = (p_ff["W1"], p_ff["b1"].reshape(1, -1), p_ff["W2"],
            p_ff["b2"].reshape(1, -1), p_lin["W"], p_lin["b"].reshape(1, -1),
            wmsg, bmsg.reshape(1, -1))
    shp = jax.ShapeDtypeStruct((n, dout), F32)
    return pl.pallas_call(
        _node_prep_body,
        grid=(grid,),
        in_specs=[pl.BlockSpec((block, din), lambda i: (i, 0))]
        + [full(a) for a in args],
        out_specs=[pl.BlockSpec((block, dout), lambda i: (i, 0))] * 2
        + [pl.BlockSpec((block, 2 * dout), lambda i: (i, 0))],
        out_shape=[shp, shp, jax.ShapeDtypeStruct((n, 2 * dout), F32)],
    )(node_feature, *args)


def _state_prep_body(gs_ref, w1, b1, w2, b2, wl, bl, out_ref):
    x = gs_ref[...]
    h = jnp.maximum(_dot(x, w1[...]) + b1[...], 0.0)
    out_ref[...] = _dot(h, w2[...]) + b2[...] + _dot(x, wl[...]) + bl[...]


def _state_prep(gs_row, p_ff, p_lin):
    b = gs_row.shape[1]
    args = (p_ff["W1"], p_ff["b1"].reshape(1, -1), p_ff["W2"],
            p_ff["b2"].reshape(1, -1), p_lin["W"], p_lin["b"].reshape(1, -1))
    return pl.pallas_call(
        _state_prep_body,
        out_shape=jax.ShapeDtypeStruct((1, b), F32),
    )(gs_row, *args)


def _update_body(x_ref, a0_ref, a1_ref, res_ref, st_ref,
                 wself, wagg, wstate, bout, wmsg, bmsg,
                 xn_ref, xmn_ref):
    x = x_ref[...]
    d = x.shape[1]
    agg = a0_ref[:, :d] + a1_ref[:, :d]
    t = (_dot(x, wself[...]) + _dot(agg, wagg[...])
         + st_ref[...] * wstate[...] + bout[...])
    xn = res_ref[...] + jnp.maximum(t, 0.0)
    xn_ref[...] = xn
    xm = _dot(xn, wmsg[...]) + bmsg[...]
    xmn_ref[...] = jnp.concatenate([xm, jnp.zeros_like(xm)], axis=1)


def _update(x, a0, a1, node_res, state_col, p, wmsg_next, bmsg_next, block):
    n, d = x.shape
    grid = n // block
    full = lambda arr: pl.BlockSpec(arr.shape, lambda i: (0,) * arr.ndim)
    args = (p["Wself"], p["Wagg"], p["wstate"].reshape(1, -1),
            p["bout"].reshape(1, -1), wmsg_next, bmsg_next.reshape(1, -1))
    blk = lambda w=d: pl.BlockSpec((block, w), lambda i: (i, 0))
    return pl.pallas_call(
        _update_body,
        grid=(grid,),
        in_specs=[blk(), blk(2 * d), blk(2 * d), blk(),
                  pl.BlockSpec((block, 1), lambda i: (i, 0))]
        + [full(a) for a in args],
        out_specs=[blk(), blk(2 * d)],
        out_shape=[jax.ShapeDtypeStruct((n, d), F32),
                   jax.ShapeDtypeStruct((n, 2 * d), F32)],
    )(x, a0, a1, node_res, state_col, *args)


def _sc_msgpass(xm, me2, src, dst, wedge):
    n, dw = xm.shape
    d = dw // 2
    e = src.shape[0]
    kc = 80
    info = plsc.get_sparse_core_info()
    nc, ns = info.num_cores, info.num_subcores
    ts = e // (nc * ns * kc)
    kh = kc // 2
    nb = 2
    cp_sub = 5
    rows_w = n // cp_sub
    zr = kc
    nz = rows_w // zr

    mesh = plsc.VectorSubcoreMesh(core_axis_name="c", subcore_axis_name="s")

    @functools.partial(
        pl.kernel,
        out_type=jax.ShapeDtypeStruct((nc, n, dw), F32),
        mesh=mesh,
        scratch_types=[
            pltpu.VMEM_SHARED((n, dw), F32),
            pltpu.VMEM((ts * kc,), jnp.int32),
            [pltpu.VMEM((kc,), jnp.int32)] * nb,
            [pltpu.VMEM((kc, dw), F32)] * nb,
            [pltpu.VMEM((kh, dw), F32)] * nb,
            pltpu.VMEM((d,), F32),
            [pltpu.SemaphoreType.DMA] * nb,
            [pltpu.SemaphoreType.DMA] * nb,
            [pltpu.SemaphoreType.DMA] * nb,
            [pltpu.SemaphoreType.DMA] * nb,
        ],
    )
    def k(xm_h, me_h, src_h, dst_h, wedge_h, out_h,
          agg_sh, srcb, dst2, gbuf, ebuf, wv, sem_g, sem_e, sem_sc,
          sem_i):
        c = lax.axis_index("c")
        s = lax.axis_index("s")

        pltpu.sync_copy(wedge_h, wv)
        w = [wv[pl.ds(j * 16, 16)] for j in range(d // 16)]
        zero = jnp.zeros((16,), F32)

        e0 = pl.multiple_of((c * ns + s) * (ts * kc), kc)
        me0 = pl.multiple_of((c * ns + s) * (ts * kh), kh)

        pltpu.sync_copy(src_h.at[pl.ds(e0, ts * kc)], srcb)

        @pl.loop(0, kc)
        def _(r):
            for j in range(dw // 16):
                gbuf[0][r, pl.ds(j * 16, 16)] = zero

        @pl.when(s < cp_sub)
        def _():
            @pl.loop(0, nz)
            def _(kz):
                r0 = pl.multiple_of(s * rows_w + kz * zr, zr)
                pltpu.sync_copy(gbuf[0], agg_sh.at[pl.ds(r0, zr)])

        plsc.subcore_barrier()

        def body(t, nbi):
            idxs = []
            for i in range(nbi):
                u = t + i
                eo = pl.multiple_of(e0 + u * kc, kc)
                idxs.append(pltpu.async_copy(dst_h.at[pl.ds(eo, kc)],
                                             dst2[i], sem_i[i]))
            loads = []
            for i in range(nbi):
                u = t + i
                so = pl.multiple_of(u * kc, kc)
                cpg = pltpu.async_copy(xm_h.at[srcb.at[pl.ds(so, kc)]],
                                       gbuf[i], sem_g[i])
                off2 = pl.multiple_of(me0 + u * kh, kh)
                cpe = pltpu.async_copy(me_h.at[pl.ds(off2, kh)], ebuf[i],
                                       sem_e[i])
                loads.append((cpg, cpe))
            stores = []
            for i in range(nbi):
                loads[i][0].wait()
                loads[i][1].wait()

                @pl.loop(0, kh)
                def _(rr):
                    for half in range(2):
                        r = 2 * rr + half
                        for j in range(d // 16):
                            gsl = pl.ds(j * 16, 16)
                            esl = pl.ds(half * d + j * 16, 16)
                            gbuf[i][r, gsl] = jnp.maximum(
                                gbuf[i][r, gsl] + ebuf[i][rr, esl] * w[j],
                                0.0)

                idxs[i].wait()
                stores.append(pltpu.async_copy(
                    gbuf[i], agg_sh.at[dst2[i]], sem_sc[i], add=True))
            for cp in stores:
                cp.wait()

        main = (ts // nb) * nb
        @pl.loop(0, main, step=nb)
        def _(t):
            body(t, nb)

        for u in range(main, ts):
            body(u, 1)

        plsc.subcore_barrier()

        @pl.when(s < cp_sub)
        def _():
            @pl.loop(0, nz)
            def _(kz):
                r0 = pl.multiple_of(s * rows_w + kz * zr, zr)
                pltpu.sync_copy(agg_sh.at[pl.ds(r0, zr)], gbuf[0])
                pltpu.sync_copy(gbuf[0], out_h.at[c].at[pl.ds(r0, zr)])

    return k(xm, me2, src, dst, wedge)


def kernel(node_feature, edge_index, edge_feature, global_state, group_size,
           params):
    n = node_feature.shape[0]
    batch = global_state.shape[0]
    group = n // batch
    convs = params["convs"]

    ef2 = edge_feature.reshape(edge_feature.shape[0] // 2,
                               2 * edge_feature.shape[1])
    me2 = _edge_prep(ef2, params["edge_ff"], params["edge_linear"],
                     block=8000)
    node_res, meta_node, xm = _node_prep(
        node_feature, params["node_ff"], params["node_linear"],
        convs[0]["Wmsg"], convs[0]["bmsg"], block=2000)
    tot_state = _state_prep(global_state.reshape(1, batch),
                            params["state_ff"], params["state_linear"])
    state_col = jnp.broadcast_to(tot_state.reshape(batch, 1, 1),
                                 (batch, group, 1)).reshape(n, 1)

    src = edge_index[0]
    dst = edge_index[1]
    nl = len(convs)
    for l, p in enumerate(convs):
        agg2 = _sc_msgpass(xm, me2, src, dst, p["wedge"])
        pn = convs[(l + 1) % nl]
        meta_node, xm = _update(meta_node, agg2[0], agg2[1], node_res,
                                state_col, p, pn["Wmsg"], pn["bmsg"],
                                block=2000)
    return meta_node
